# baseline (speedup 1.0000x reference)
import functools

import jax
import jax.numpy as jnp
from jax import lax
from jax.experimental import pallas as pl
from jax.experimental.pallas import tpu as pltpu
from jax.experimental.pallas import tpu_sc as plsc

EOS_ID = 49407
SEQ = 77
DIM = 512
NSEQ = 4096
LANES = 16
NC = 2
NS = 16
NW = NC * NS

TOK = NSEQ * SEQ
CH = 64
NCHUNK = TOK // CH
CPW = NCHUNK // NW
PAIRS = CPW // 2
IDX_ROWS = 160


def _emb_body(ids2d, table, pos, out, idx_all, pos_v, rows,
              gsem0, gsem1, wsem0, wsem1):
    wid = lax.axis_index("s") * NC + lax.axis_index("c")
    c0w = wid * CPW
    r0a = 8 * (c0w // 8)
    off = c0w - r0a
    pltpu.sync_copy(pos, pos_v)
    pltpu.sync_copy(ids2d.at[pl.ds(r0a, IDX_ROWS)], idx_all)

    def start_gather(buf, sem, c_local):
        pltpu.async_copy(table.at[idx_all.at[off + c_local]], rows.at[buf], sem)

    def wait_gather(buf, sem):
        pltpu.make_async_copy(table.at[pl.ds(0, CH)], rows.at[buf], sem).wait()

    def start_write(buf, sem, c_local):
        pltpu.async_copy(rows.at[buf], out.at[c0w + c_local], sem)

    def wait_write(sem):
        pltpu.make_async_copy(rows.at[0], out.at[0], sem).wait()

    def add_pos(buf, c_local):
        p0 = lax.rem((c0w + c_local) * CH, SEQ)

        def row_body(r, carry):
            p = lax.rem(p0 + r, SEQ)
            for cc in range(DIM // LANES):
                sl = pl.ds(cc * LANES, LANES)
                rows[buf, r, sl] = rows[buf, r, sl] + pos_v[p, sl]
            return carry

        lax.fori_loop(0, CH, row_body, 0)

    start_gather(0, gsem0, 0)

    def pair_body(s2, carry):
        cl0 = 2 * s2
        cl1 = 2 * s2 + 1
        wait_gather(0, gsem0)

        @pl.when(s2 > 0)
        def _():
            wait_write(wsem1)

        start_gather(1, gsem1, cl1)
        add_pos(0, cl0)
        start_write(0, wsem0, cl0)
        wait_gather(1, gsem1)
        wait_write(wsem0)

        @pl.when(s2 < PAIRS - 1)
        def _():
            start_gather(0, gsem0, cl0 + 2)

        add_pos(1, cl1)
        start_write(1, wsem1, cl1)
        return carry

    lax.fori_loop(0, PAIRS, pair_body, 0)
    wait_write(wsem1)


def _mask_body(ids_ref, out_ref):
    out_ref[...] = ids_ref[...] == EOS_ID


def kernel(input_ids, embedding_table, positional_embedding):
    ids2d = input_ids.reshape(NCHUNK, CH)
    mesh = plsc.VectorSubcoreMesh(core_axis_name="c", subcore_axis_name="s")
    emb = functools.partial(
        pl.kernel,
        mesh=mesh,
        out_type=jax.ShapeDtypeStruct((NCHUNK, CH, DIM), jnp.float32),
        scratch_types=[
            pltpu.VMEM((IDX_ROWS, CH), jnp.int32),
            pltpu.VMEM((SEQ, DIM), jnp.float32),
            pltpu.VMEM((2, CH, DIM), jnp.float32),
            pltpu.SemaphoreType.DMA,
            pltpu.SemaphoreType.DMA,
            pltpu.SemaphoreType.DMA,
            pltpu.SemaphoreType.DMA,
        ],
    )(_emb_body)
    tokens = emb(ids2d, embedding_table, positional_embedding)
    tokens = tokens.reshape(NSEQ, SEQ, DIM)
    mask = pl.pallas_call(
        _mask_body,
        out_shape=jax.ShapeDtypeStruct((NSEQ, SEQ), jnp.bool_),
    )(input_ids)
    return (tokens, mask)


# speedup vs baseline: 1.4633x; 1.4633x over previous
import functools

import jax
import jax.numpy as jnp
from jax import lax
from jax.experimental import pallas as pl
from jax.experimental.pallas import tpu as pltpu
from jax.experimental.pallas import tpu_sc as plsc

EOS_ID = 49407
SEQ = 77
SEQ_PAD = 80
MAIN = 72
TAIL = 8
DIM = 512
NSEQ = 4096
LANES = 16
NC = 2
NS = 16
NW = NC * NS
SPW = NSEQ // NW
PAIRS = SPW // 2


def _emb_body(ids_hbm, table, pos, out, idx2, pos_v, seqbuf, side,
              gsem0, gsem1, wsem0, wsem1, isem):
    wid = lax.axis_index("s") * NC + lax.axis_index("c")
    base = wid * SPW
    pltpu.sync_copy(pos, pos_v)
    pltpu.sync_copy(ids_hbm.at[base], idx2.at[0])
    pltpu.sync_copy(ids_hbm.at[base + 1], idx2.at[1])

    def start_gathers(buf, q, sem):
        pltpu.async_copy(table.at[idx2.at[q, pl.ds(0, MAIN)]],
                         seqbuf.at[buf, pl.ds(0, MAIN)], sem)
        pltpu.async_copy(table.at[idx2.at[q, pl.ds(MAIN, TAIL)]], side, sem)

    def wait_gathers(buf, sem):
        pltpu.make_async_copy(table.at[pl.ds(0, MAIN)],
                              seqbuf.at[buf, pl.ds(0, MAIN)], sem).wait()
        pltpu.make_async_copy(table.at[pl.ds(0, TAIL)], side, sem).wait()

    def start_write(buf, sem, s):
        pltpu.async_copy(seqbuf.at[buf], out.at[base + s], sem)

    def wait_write(sem):
        pltpu.make_async_copy(seqbuf.at[0], out.at[0], sem).wait()

    def prefetch_idx(q, s):
        pltpu.async_copy(ids_hbm.at[base + s], idx2.at[q], isem)

    def wait_idx():
        pltpu.make_async_copy(ids_hbm.at[0], idx2.at[0], isem).wait()

    def tailpass(buf):
        def row_body(r, carry):
            for cc in range(DIM // LANES):
                sl = pl.ds(cc * LANES, LANES)
                seqbuf[buf, MAIN + r, sl] = side[r, sl] + pos_v[MAIN + r, sl]
            return carry

        lax.fori_loop(0, SEQ - MAIN, row_body, 0)

    def mainpass(buf):
        def row_body(r, carry):
            for cc in range(DIM // LANES):
                sl = pl.ds(cc * LANES, LANES)
                seqbuf[buf, r, sl] = seqbuf[buf, r, sl] + pos_v[r, sl]
            return carry

        lax.fori_loop(0, MAIN, row_body, 0)

    start_gathers(0, 0, gsem0)

    def pair_body(s2, carry):
        s0 = 2 * s2
        s1 = s0 + 1
        wait_gathers(0, gsem0)
        tailpass(0)

        @pl.when(s2 > 0)
        def _():
            wait_write(wsem1)
            wait_idx()

        start_gathers(1, 1, gsem1)

        @pl.when(s2 < PAIRS - 1)
        def _():
            prefetch_idx(0, s0 + 2)

        mainpass(0)
        start_write(0, wsem0, s0)
        wait_gathers(1, gsem1)
        tailpass(1)
        wait_write(wsem0)

        @pl.when(s2 < PAIRS - 1)
        def _():
            wait_idx()
            start_gathers(0, 0, gsem0)
            prefetch_idx(1, s1 + 2)

        mainpass(1)
        start_write(1, wsem1, s1)
        return carry

    lax.fori_loop(0, PAIRS, pair_body, 0)
    wait_write(wsem1)


def _mask_body(ids_ref, out_ref):
    out_ref[...] = ids_ref[...] == EOS_ID


def kernel(input_ids, embedding_table, positional_embedding):
    ids80 = jnp.pad(input_ids, ((0, 0), (0, SEQ_PAD - SEQ)))
    mesh = plsc.VectorSubcoreMesh(core_axis_name="c", subcore_axis_name="s")
    emb = functools.partial(
        pl.kernel,
        mesh=mesh,
        out_type=jax.ShapeDtypeStruct((NSEQ, SEQ, DIM), jnp.float32),
        scratch_types=[
            pltpu.VMEM((2, SEQ_PAD), jnp.int32),
            pltpu.VMEM((SEQ, DIM), jnp.float32),
            pltpu.VMEM((2, SEQ, DIM), jnp.float32),
            pltpu.VMEM((TAIL, DIM), jnp.float32),
            pltpu.SemaphoreType.DMA,
            pltpu.SemaphoreType.DMA,
            pltpu.SemaphoreType.DMA,
            pltpu.SemaphoreType.DMA,
            pltpu.SemaphoreType.DMA,
        ],
    )(_emb_body)
    tokens = emb(ids80, embedding_table, positional_embedding)
    mask = pl.pallas_call(
        _mask_body,
        out_shape=jax.ShapeDtypeStruct((NSEQ, SEQ), jnp.bool_),
    )(input_ids)
    return (tokens, mask)


# speedup vs baseline: 5.6374x; 3.8526x over previous
import functools

import jax
import jax.numpy as jnp
from jax import lax
from jax.experimental import pallas as pl
from jax.experimental.pallas import tpu as pltpu
from jax.experimental.pallas import tpu_sc as plsc

EOS_ID = 49407
SEQ = 77
DIM = 512
NSEQ = 4096
LANES = 16
NC = 2
NS = 16
NW = NC * NS
SPW = NSEQ // NW
NB = 4
QTR = SPW // NB


def _emb_body(ids_t, table, pos, out, idx, pos_v, buf,
              gs0, gs1, gs2, gs3, ws0, ws1, ws2, ws3):
    gsem = (gs0, gs1, gs2, gs3)
    wsem = (ws0, ws1, ws2, ws3)
    wid = lax.axis_index("s") * NC + lax.axis_index("c")
    base = wid * SPW
    pltpu.sync_copy(pos, pos_v)
    pltpu.sync_copy(ids_t.at[:, pl.ds(base, SPW)], idx)

    def start_gather(h, p):
        pltpu.async_copy(table.at[idx.at[p, pl.ds(h * QTR, QTR)]],
                         buf.at[h], gsem[h])

    def wait_gather(h):
        pltpu.make_async_copy(table.at[pl.ds(0, QTR)], buf.at[h],
                              gsem[h]).wait()

    def start_write(h, p):
        pltpu.async_copy(buf.at[h], out.at[p, pl.ds(base + h * QTR, QTR)],
                         wsem[h])

    def wait_write(h):
        pltpu.make_async_copy(buf.at[0], out.at[0, pl.ds(0, QTR)],
                              wsem[h]).wait()

    def addpass(h, p):
        for cb in range(DIM // LANES // 8):
            pvs = [pos_v[p, pl.ds((cb * 8 + j) * LANES, LANES)]
                   for j in range(8)]

            def row_body(r, carry):
                for j in range(8):
                    sl = pl.ds((cb * 8 + j) * LANES, LANES)
                    buf[h, r, sl] = buf[h, r, sl] + pvs[j]
                return carry

            lax.fori_loop(0, QTR, row_body, 0)

    start_gather(0, 0)
    start_gather(1, 0)
    start_gather(2, 0)

    def ploop(p, carry):
        for h in range(NB):
            wait_gather(h)
            addpass(h, p)
            start_write(h, p)
            hp = (h + 3) % NB
            if h == 0:
                @pl.when(p > 0)
                def _():
                    wait_write(hp)

                start_gather(hp, p)
            else:
                wait_write(hp)

                @pl.when(p < SEQ - 1)
                def _():
                    start_gather(hp, p + 1)

        return carry

    lax.fori_loop(0, SEQ, ploop, 0)
    wait_write(NB - 1)


def _mask_body(ids_ref, out_ref):
    out_ref[...] = ids_ref[...] == EOS_ID


def kernel(input_ids, embedding_table, positional_embedding):
    ids_t = jnp.transpose(input_ids)
    mesh = plsc.VectorSubcoreMesh(core_axis_name="c", subcore_axis_name="s")
    emb = functools.partial(
        pl.kernel,
        mesh=mesh,
        out_type=jax.ShapeDtypeStruct((SEQ, NSEQ, DIM), jnp.float32),
        scratch_types=[
            pltpu.VMEM((SEQ, SPW), jnp.int32),
            pltpu.VMEM((SEQ, DIM), jnp.float32),
            pltpu.VMEM((NB, QTR, DIM), jnp.float32),
            pltpu.SemaphoreType.DMA,
            pltpu.SemaphoreType.DMA,
            pltpu.SemaphoreType.DMA,
            pltpu.SemaphoreType.DMA,
            pltpu.SemaphoreType.DMA,
            pltpu.SemaphoreType.DMA,
            pltpu.SemaphoreType.DMA,
            pltpu.SemaphoreType.DMA,
        ],
    )(_emb_body)
    tokens_t = emb(ids_t, embedding_table, positional_embedding)
    tokens = jnp.transpose(tokens_t, (1, 0, 2))
    mask = pl.pallas_call(
        _mask_body,
        out_shape=jax.ShapeDtypeStruct((NSEQ, SEQ), jnp.bool_),
    )(input_ids)
    return (tokens, mask)
